# baseline (speedup 1.0000x reference)
import functools
import jax
import jax.numpy as jnp
from jax import lax
from jax.experimental import pallas as pl
from jax.experimental.pallas import tpu as pltpu, tpu_sc as plsc

B, L, D = 4096, 200, 64
V_TAB = 1000

_info = plsc.get_sparse_core_info()
NC, NS, LN = _info.num_cores, _info.num_subcores, _info.num_lanes
NW = NC * NS
PER_W = (B * L) // NW
ROWS_W = B // NW
NVEC = PER_W // LN
CHUNK = 128
NCHUNK = PER_W // CHUNK


def _body(mask_hbm, table_hbm, out_hbm, mask_v, cum_v, base_v, rows_v, tab_sh, sem):
    sid = lax.axis_index("s")
    wid = sid * NC + lax.axis_index("c")
    flat0 = wid * PER_W

    @pl.when(sid == 0)
    def _():
        pltpu.sync_copy(table_hbm, tab_sh)

    pltpu.sync_copy(mask_hbm.at[pl.ds(flat0, PER_W)], mask_v)

    iota = lax.iota(jnp.int32, LN)

    def cum_body(v, carry):
        x = mask_v[pl.ds(v * LN, LN)]
        cum_v[pl.ds(v * LN, LN)] = plsc.cumsum(x) + carry
        return carry + jnp.sum(x)

    lax.fori_loop(0, NVEC, cum_body, jnp.int32(0))

    for k in range(ROWS_W // LN):
        r = iota + k * LN
        idx = jnp.maximum(r * L - 1, 0)
        g = plsc.load_gather(cum_v, [idx])
        base_v[pl.ds(k * LN, LN)] = jnp.where(r == 0, 0, g)

    def pos_body(v, _):
        x = mask_v[pl.ds(v * LN, LN)]
        c = cum_v[pl.ds(v * LN, LN)]
        b = (iota + v * LN) // L
        base = plsc.load_gather(base_v, [b])
        mask_v[pl.ds(v * LN, LN)] = (c - base) * x
        return 0

    lax.fori_loop(0, NVEC, pos_body, jnp.int32(0))

    plsc.subcore_barrier()

    def gat_body(j, _):
        idx_ref = mask_v.at[pl.ds(j * CHUNK, CHUNK)]
        pltpu.async_copy(tab_sh.at[idx_ref], rows_v, sem).wait()
        pltpu.sync_copy(rows_v, out_hbm.at[pl.ds(flat0 + j * CHUNK, CHUNK)])
        return 0

    lax.fori_loop(0, NCHUNK, gat_body, jnp.int32(0))


@functools.partial(jax.jit, donate_argnums=())
def _run(mask_flat, table):
    kern = pl.kernel(
        _body,
        out_type=jax.ShapeDtypeStruct((B * L, D), jnp.float32),
        mesh=plsc.VectorSubcoreMesh(core_axis_name="c", subcore_axis_name="s"),
        scratch_types=[
            pltpu.VMEM((PER_W,), jnp.int32),
            pltpu.VMEM((PER_W,), jnp.int32),
            pltpu.VMEM((ROWS_W,), jnp.int32),
            pltpu.VMEM((CHUNK, D), jnp.float32),
            pltpu.VMEM_SHARED((V_TAB, D), jnp.float32),
            pltpu.SemaphoreType.DMA,
        ],
        compiler_params=pltpu.CompilerParams(
            needs_layout_passes=False, use_tc_tiling_on_sc=False
        ),
    )
    return kern(mask_flat, table)


def kernel(input, mask, table):
    del input
    out = _run(mask.reshape(-1).astype(jnp.int32), table)
    return out.reshape(B, L, D)


# speedup vs baseline: 1.0028x; 1.0028x over previous
import functools
import jax
import jax.numpy as jnp
from jax import lax
from jax.experimental import pallas as pl
from jax.experimental.pallas import tpu as pltpu, tpu_sc as plsc

B, L, D = 4096, 200, 64
V_TAB = 1000

_info = plsc.get_sparse_core_info()
NC, NS, LN = _info.num_cores, _info.num_subcores, _info.num_lanes
NW = NC * NS
PER_W = (B * L) // NW
ROWS_W = B // NW
NVEC = PER_W // LN
CHUNK = 128
NCHUNK = PER_W // CHUNK


def _body(mask_hbm, table_hbm, out_hbm, mask_v, rows_v, tab_sh, sem):
    sid = lax.axis_index("s")
    wid = sid * NC + lax.axis_index("c")
    flat0 = wid * PER_W

    @pl.when(sid == 0)
    def _():
        pltpu.sync_copy(table_hbm, tab_sh)

    pltpu.sync_copy(mask_hbm.at[pl.ds(flat0, PER_W)], mask_v)

    iota_l = lax.iota(jnp.int32, LN) * L
    zeros = jnp.zeros((LN,), jnp.int32)

    def cum_body(l, accs):
        new = []
        for g in range(ROWS_W // LN):
            idx = iota_l + (l + g * (LN * L))
            x = plsc.load_gather(mask_v, [idx])
            a = accs[g] + x
            plsc.store_scatter(mask_v, [idx], a * x)
            new.append(a)
        return tuple(new)

    lax.fori_loop(0, L, cum_body, (zeros,) * (ROWS_W // LN))

    plsc.subcore_barrier()

    def gat_body(j, _):
        idx_ref = mask_v.at[pl.ds(j * CHUNK, CHUNK)]
        pltpu.async_copy(tab_sh.at[idx_ref], rows_v, sem).wait()
        pltpu.sync_copy(rows_v, out_hbm.at[pl.ds(flat0 + j * CHUNK, CHUNK)])
        return 0

    lax.fori_loop(0, NCHUNK, gat_body, jnp.int32(0))


@functools.partial(jax.jit, donate_argnums=())
def _run(mask_flat, table):
    kern = pl.kernel(
        _body,
        out_type=jax.ShapeDtypeStruct((B * L, D), jnp.float32),
        mesh=plsc.VectorSubcoreMesh(core_axis_name="c", subcore_axis_name="s"),
        scratch_types=[
            pltpu.VMEM((PER_W,), jnp.int32),
            pltpu.VMEM((CHUNK, D), jnp.float32),
            pltpu.VMEM_SHARED((V_TAB, D), jnp.float32),
            pltpu.SemaphoreType.DMA,
        ],
        compiler_params=pltpu.CompilerParams(
            needs_layout_passes=False, use_tc_tiling_on_sc=False
        ),
    )
    return kern(mask_flat, table)


def kernel(input, mask, table):
    del input
    out = _run(mask.reshape(-1).astype(jnp.int32), table)
    return out.reshape(B, L, D)


# speedup vs baseline: 1.0323x; 1.0294x over previous
import functools
import jax
import jax.numpy as jnp
from jax import lax
from jax.experimental import pallas as pl
from jax.experimental.pallas import tpu as pltpu, tpu_sc as plsc

B, L, D = 4096, 200, 64
V_TAB = 1000

_info = plsc.get_sparse_core_info()
NC, NS, LN = _info.num_cores, _info.num_subcores, _info.num_lanes
NW = NC * NS
PER_W = (B * L) // NW
ROWS_W = B // NW
GRP = ROWS_W // LN


def _body(mask_hbm, table_hbm, out_hbm, mask_v, rows0, rows1, tab_sh,
          sg0, sg1, sw0, sw1):
    sid = lax.axis_index("s")
    wid = sid * NC + lax.axis_index("c")
    flat0 = wid * PER_W
    row0 = wid * ROWS_W

    @pl.when(sid == 0)
    def _():
        pltpu.sync_copy(table_hbm, tab_sh)

    pltpu.sync_copy(mask_hbm.at[pl.ds(flat0, PER_W)], mask_v)

    iota_l = lax.iota(jnp.int32, LN) * L
    zeros = jnp.zeros((LN,), jnp.int32)

    def cum_body(l, accs):
        new = []
        for g in range(GRP):
            idx = iota_l + (l + g * (LN * L))
            x = plsc.load_gather(mask_v, [idx])
            a = accs[g] + x
            plsc.store_scatter(mask_v, [idx], a * x)
            new.append(a)
        return tuple(new)

    lax.fori_loop(0, L, cum_body, (zeros,) * GRP)

    plsc.subcore_barrier()

    def gstart(b, buf, sem):
        idx_ref = mask_v.at[pl.ds(b * L, L)]
        pltpu.async_copy(tab_sh.at[idx_ref], buf, sem)

    def gwait(buf, sem):
        pltpu.make_async_copy(tab_sh.at[mask_v.at[pl.ds(0, L)]], buf, sem).wait()

    def wstart(b, buf, sem):
        pltpu.async_copy(buf, out_hbm.at[row0 + b], sem)

    def wwait(b, buf, sem):
        pltpu.make_async_copy(buf, out_hbm.at[row0 + b], sem).wait()

    gstart(0, rows0, sg0)

    def pair_body(i, _):
        b0 = 2 * i

        @pl.when(i > 0)
        def _():
            wwait(b0 - 1, rows1, sw1)

        gstart(b0 + 1, rows1, sg1)
        gwait(rows0, sg0)
        wstart(b0, rows0, sw0)
        wwait(b0, rows0, sw0)

        @pl.when(b0 + 2 < ROWS_W)
        def _():
            gstart(b0 + 2, rows0, sg0)

        gwait(rows1, sg1)
        wstart(b0 + 1, rows1, sw1)
        return 0

    lax.fori_loop(0, ROWS_W // 2, pair_body, jnp.int32(0))
    wwait(ROWS_W - 1, rows1, sw1)


@functools.partial(jax.jit, donate_argnums=())
def _run(mask_flat, table):
    kern = pl.kernel(
        _body,
        out_type=jax.ShapeDtypeStruct((B, L, D), jnp.float32),
        mesh=plsc.VectorSubcoreMesh(core_axis_name="c", subcore_axis_name="s"),
        scratch_types=[
            pltpu.VMEM((PER_W,), jnp.int32),
            pltpu.VMEM((L, D), jnp.float32),
            pltpu.VMEM((L, D), jnp.float32),
            pltpu.VMEM_SHARED((V_TAB, D), jnp.float32),
            pltpu.SemaphoreType.DMA,
            pltpu.SemaphoreType.DMA,
            pltpu.SemaphoreType.DMA,
            pltpu.SemaphoreType.DMA,
        ],
        compiler_params=pltpu.CompilerParams(
            needs_layout_passes=False, use_tc_tiling_on_sc=False
        ),
    )
    return kern(mask_flat, table)


def kernel(input, mask, table):
    del input
    return _run(mask.reshape(-1).astype(jnp.int32), table)


# speedup vs baseline: 1.3471x; 1.3050x over previous
import functools
import jax
import jax.numpy as jnp
from jax import lax
from jax.experimental import pallas as pl
from jax.experimental.pallas import tpu as pltpu, tpu_sc as plsc

B, L, D = 4096, 200, 64
V_TAB = 1000

_info = plsc.get_sparse_core_info()
NC, NS, LN = _info.num_cores, _info.num_subcores, _info.num_lanes
NW = NC * NS
PER_W = (B * L) // NW
ROWS_W = B // NW
GRP = ROWS_W // LN


def _body(mask_hbm, table_hbm, out_hbm, mask_v, rows0, rows1, tab_sh,
          sg0, sg1, sw0, sw1):
    sid = lax.axis_index("s")
    wid = sid * NC + lax.axis_index("c")
    flat0 = wid * PER_W
    row0 = wid * ROWS_W

    @pl.when(sid == 0)
    def _():
        pltpu.sync_copy(table_hbm, tab_sh)

    pltpu.sync_copy(mask_hbm.at[pl.ds(flat0, PER_W)], mask_v)

    iota_l = lax.iota(jnp.int32, LN) * L
    zeros = jnp.zeros((LN,), jnp.int32)

    def cum_body(l, accs):
        new = []
        for g in range(GRP):
            idx = iota_l + (l + g * (LN * L))
            x = plsc.load_gather(mask_v, [idx])
            a = accs[g] + x
            plsc.store_scatter(mask_v, [idx], a * x)
            new.append(a)
        return tuple(new)

    lax.fori_loop(0, L, cum_body, (zeros,) * GRP)

    plsc.subcore_barrier()

    def gstart(b, buf, sem):
        idx_ref = mask_v.at[pl.ds(b * L, L)]
        pltpu.async_copy(tab_sh.at[idx_ref], buf, sem)

    def gwait(buf, sem):
        pltpu.make_async_copy(tab_sh.at[mask_v.at[pl.ds(0, L)]], buf, sem).wait()

    def wstart(b, buf, sem):
        pltpu.async_copy(buf, out_hbm.at[row0 + b], sem)

    def wwait(b, buf, sem):
        pltpu.make_async_copy(buf, out_hbm.at[row0 + b], sem).wait()

    gstart(0, rows0, sg0)

    def pair_body(i, _):
        b0 = 2 * i

        @pl.when(i > 0)
        def _():
            wwait(b0 - 1, rows1, sw1)

        gstart(b0 + 1, rows1, sg1)
        gwait(rows0, sg0)
        wstart(b0, rows0, sw0)
        wwait(b0, rows0, sw0)

        @pl.when(b0 + 2 < ROWS_W)
        def _():
            gstart(b0 + 2, rows0, sg0)

        gwait(rows1, sg1)
        wstart(b0 + 1, rows1, sw1)
        return 0

    lax.fori_loop(0, ROWS_W // 2, pair_body, jnp.int32(0))
    wwait(ROWS_W - 1, rows1, sw1)


@functools.partial(jax.jit, donate_argnums=())
def _run(mask_flat, table):
    kern = pl.kernel(
        _body,
        out_type=jax.ShapeDtypeStruct((B, L, D), jnp.float32),
        mesh=plsc.VectorSubcoreMesh(core_axis_name="c", subcore_axis_name="s"),
        scratch_types=[
            pltpu.VMEM((PER_W,), jnp.int32),
            pltpu.VMEM((L, D), jnp.float32),
            pltpu.VMEM((L, D), jnp.float32),
            pltpu.VMEM_SHARED((V_TAB, D), jnp.float32),
            pltpu.SemaphoreType.DMA,
            pltpu.SemaphoreType.DMA,
            pltpu.SemaphoreType.DMA,
            pltpu.SemaphoreType.DMA,
        ],
        compiler_params=pltpu.CompilerParams(needs_layout_passes=False),
    )
    return kern(mask_flat, table)


def kernel(input, mask, table):
    del input
    return _run(mask.reshape(-1).astype(jnp.int32), table)
